# baseline (speedup 1.0000x reference)
import functools
import math

import jax
import jax.numpy as jnp
from jax import lax
from jax.experimental import pallas as pl
from jax.experimental.pallas import tpu as pltpu
from jax.experimental.pallas import tpu_sc as plsc

_N = 10000
_E = 320000
_H = 128
_NH = 4
_DK = _H // _NH

_NC = 2
_NS = 16
_NW = _NC * _NS
_CH = 40
_NP = 10240
_RPS = _NP // _NS

_NCK = 5
_EC = _E // _NCK
_EPWC = _EC // _NW
_NSUBC = _EPWC // _CH
_NPAIR = _NSUBC // 2

_BE = 512
_EBLK = _EC // _BE
_BN = 1000

_F32 = jnp.float32


_U32 = jnp.uint32


def _pack16(lo, hi):
    lo_u = jax.lax.bitcast_convert_type(lo, _U32)
    hi_u = jax.lax.bitcast_convert_type(hi, _U32)
    word = (lo_u >> _U32(16)) | (hi_u & _U32(0xFFFF0000))
    return jax.lax.bitcast_convert_type(word, _F32)


def _unpack16(word):
    u = jax.lax.bitcast_convert_type(word, _U32)
    lo = jax.lax.bitcast_convert_type(u << _U32(16), _F32)
    hi = jax.lax.bitcast_convert_type(u & _U32(0xFFFF0000), _F32)
    return lo, hi


def _proj_body(x_ref, wd_ref, ws_ref, td_ref, ts_ref):
    xv = x_ref[...]
    a = jnp.dot(xv, wd_ref[:, :_H], preferred_element_type=_F32)
    q = jnp.dot(xv, wd_ref[:, _H:], preferred_element_type=_F32)
    b = jnp.dot(xv, ws_ref[:, :_H], preferred_element_type=_F32)
    k = jnp.dot(xv, ws_ref[:, _H:], preferred_element_type=_F32)
    td_ref[...] = _pack16(a, q)
    ts_ref[...] = _pack16(b, k)


def _proj(x, wd, ws):
    return pl.pallas_call(
        _proj_body,
        grid=(_N // _BN,),
        in_specs=[
            pl.BlockSpec((_BN, _H), lambda i: (i, 0)),
            pl.BlockSpec((_H, 2 * _H), lambda i: (0, 0)),
            pl.BlockSpec((_H, 2 * _H), lambda i: (0, 0)),
        ],
        out_specs=[
            pl.BlockSpec((_BN, _H), lambda i: (i, 0)),
            pl.BlockSpec((_BN, _H), lambda i: (i, 0)),
        ],
        out_shape=[
            jax.ShapeDtypeStruct((_N, _H), _F32),
            jax.ShapeDtypeStruct((_N, _H), _F32),
        ],
    )(x, wd, ws)


_sc_mesh = plsc.VectorSubcoreMesh(core_axis_name="c", subcore_axis_name="s")


@functools.partial(
    pl.kernel,
    mesh=_sc_mesh,
    out_type=[
        jax.ShapeDtypeStruct((_EC, _H), _F32),
        jax.ShapeDtypeStruct((_EC, _H), _F32),
    ],
    scratch_types=[
        pltpu.VMEM((_CH,), jnp.int32),
        pltpu.VMEM((_CH,), jnp.int32),
        pltpu.VMEM((_CH,), jnp.int32),
        pltpu.VMEM((_CH,), jnp.int32),
        pltpu.VMEM((_CH, _H), _F32),
        pltpu.VMEM((_CH, _H), _F32),
        pltpu.VMEM((_CH, _H), _F32),
        pltpu.VMEM((_CH, _H), _F32),
        pltpu.SemaphoreType.DMA,
        pltpu.SemaphoreType.DMA,
        pltpu.SemaphoreType.DMA,
        pltpu.SemaphoreType.DMA,
        pltpu.SemaphoreType.DMA,
        pltpu.SemaphoreType.DMA,
        pltpu.SemaphoreType.DMA,
        pltpu.SemaphoreType.DMA,
    ],
)
def _sc_gather(td_hbm, ts_hbm, dst_hbm, src_hbm, gd_out, gs_out,
               ixd0, ixd1, ixs0, ixs1, rd0, rd1, rs0, rs1,
               sgd0, sgd1, sgs0, sgs1, sod0, sod1, sos0, sos1):
    wid = lax.axis_index("s") * _NC + lax.axis_index("c")
    base0 = wid * _EPWC
    ixd = (ixd0, ixd1)
    ixs = (ixs0, ixs1)
    rd = (rd0, rd1)
    rs = (rs0, rs1)
    sgd = (sgd0, sgd1)
    sgs = (sgs0, sgs1)
    sod = (sod0, sod1)
    sos = (sos0, sos1)

    def fire(i, p):
        b = base0 + i * _CH
        pltpu.sync_copy(dst_hbm.at[pl.ds(b, _CH)], ixd[p])
        pltpu.sync_copy(src_hbm.at[pl.ds(b, _CH)], ixs[p])
        pltpu.async_copy(td_hbm.at[ixd[p]], rd[p], sgd[p])
        pltpu.async_copy(ts_hbm.at[ixs[p]], rs[p], sgs[p])

    def wait_gather(p):
        pltpu.make_async_copy(td_hbm.at[ixd[p]], rd[p], sgd[p]).wait()
        pltpu.make_async_copy(ts_hbm.at[ixs[p]], rs[p], sgs[p]).wait()

    def store(i, p):
        b = base0 + i * _CH
        pltpu.async_copy(rd[p], gd_out.at[pl.ds(b, _CH)], sod[p])
        pltpu.async_copy(rs[p], gs_out.at[pl.ds(b, _CH)], sos[p])

    def wait_store(i, p):
        b = base0 + i * _CH
        pltpu.make_async_copy(rd[p], gd_out.at[pl.ds(b, _CH)], sod[p]).wait()
        pltpu.make_async_copy(rs[p], gs_out.at[pl.ds(b, _CH)], sos[p]).wait()

    fire(0, 0)

    def body(k, carry):
        i0 = 2 * k
        fire(i0 + 1, 1)
        wait_gather(0)
        store(i0, 0)
        wait_store(i0, 0)

        @pl.when(k < _NPAIR - 1)
        def _():
            fire(i0 + 2, 0)

        wait_gather(1)
        store(i0 + 1, 1)
        wait_store(i0 + 1, 1)
        return carry

    lax.fori_loop(0, _NPAIR, body, 0)


def _edge_body(gd_ref, gs_ref, ea_ref, w1c_ref, b1_ref, w2_ref, b2_ref,
               g_ref, b_ref, wv_ref, seg_ref, exp_ref,
               eout_ref, pout_ref, wout_ref):
    bf = jnp.bfloat16
    ea = ea_ref[...]
    ad, qd = _unpack16(gd_ref[...])
    bs, ks = _unpack16(gs_ref[...])
    h1 = jnp.dot(ea.astype(bf), w1c_ref[...].astype(bf),
                 preferred_element_type=_F32)
    h1 = jnp.maximum(h1 + ad + bs + b1_ref[...], 0.0)
    h2 = jnp.dot(h1.astype(bf), w2_ref[...].astype(bf),
                 preferred_element_type=_F32) + b2_ref[...]
    mu = jnp.mean(h2, axis=-1, keepdims=True)
    dcen = h2 - mu
    var = jnp.mean(dcen * dcen, axis=-1, keepdims=True)
    e_new = g_ref[...] * dcen / jnp.sqrt(var + 1e-5) + b_ref[...]
    eout_ref[...] = ea + e_new
    prod = qd * ks
    w16 = jnp.exp(jnp.dot(prod.astype(bf), seg_ref[...].astype(bf),
                          preferred_element_type=_F32))
    wb = jnp.dot(w16.astype(bf), exp_ref[...].astype(bf),
                 preferred_element_type=_F32)
    wout_ref[...] = wb
    v = jnp.dot(e_new.astype(bf), wv_ref[...].astype(bf),
                preferred_element_type=_F32)
    pout_ref[...] = v * wb


def _edge(gd, gs, ea, w1c, b1, w2, b2, g, b, wv, seg16, exp16, ck):
    cmat = lambda shape: pl.BlockSpec(shape, lambda i: (0, 0))
    return pl.pallas_call(
        _edge_body,
        grid=(_EBLK,),
        in_specs=[
            pl.BlockSpec((_BE, _H), lambda i: (i, 0)),
            pl.BlockSpec((_BE, _H), lambda i: (i, 0)),
            pl.BlockSpec((_BE, _H), lambda i, c=ck: (i + c * _EBLK, 0)),
            cmat((_H, _H)), cmat((1, _H)), cmat((_H, _H)), cmat((1, _H)),
            cmat((1, _H)), cmat((1, _H)), cmat((_H, _H)),
            cmat((_H, 16)), cmat((16, _H)),
        ],
        out_specs=[
            pl.BlockSpec((_BE, _H), lambda i: (i, 0)),
            pl.BlockSpec((_BE, _H), lambda i: (i, 0)),
            pl.BlockSpec((_BE, _H), lambda i: (i, 0)),
        ],
        out_shape=[
            jax.ShapeDtypeStruct((_EC, _H), _F32),
            jax.ShapeDtypeStruct((_EC, _H), _F32),
            jax.ShapeDtypeStruct((_EC, _H), _F32),
        ],
    )(gd, gs, ea, w1c, b1, w2, b2, g, b, wv, seg16, exp16)


@functools.partial(
    pl.kernel,
    mesh=_sc_mesh,
    out_type=[
        jax.ShapeDtypeStruct((_NC * _NP, _H), _F32),
        jax.ShapeDtypeStruct((_NC * _NP, _H), _F32),
    ],
    scratch_types=[
        pltpu.VMEM((_CH,), jnp.int32),
        pltpu.VMEM((_CH,), jnp.int32),
        pltpu.VMEM((_CH, _H), _F32),
        pltpu.VMEM((_CH, _H), _F32),
        pltpu.SemaphoreType.DMA,
        pltpu.SemaphoreType.DMA,
        pltpu.VMEM_SHARED((_NP, _H), _F32),
    ],
)
def _sc_scatter(p0, p1, p2, p3, p4, w0, w1, w2, w3, w4, dst_hbm,
                macc_out, wacc_out, ix0, ix1, b0, b1, sl0, sl1, acc_sh):
    cid = lax.axis_index("c")
    sid = lax.axis_index("s")
    wid = sid * _NC + cid
    slab = sid * _RPS
    nsub = _RPS // _CH
    obase = cid * _NP + slab
    zv = jnp.zeros((16,), _F32)
    ix = (ix0, ix1)
    bb = (b0, b1)
    sl = (sl0, sl1)

    def zero_vmem():
        def zbody(r, carry):
            for cc in range(_H // 16):
                b0[r, pl.ds(cc * 16, 16)] = zv
            return carry

        lax.fori_loop(0, _CH, zbody, 0)

    def zero_slab():
        def ibody(j, carry):
            pltpu.sync_copy(b0, acc_sh.at[pl.ds(slab + j * _CH, _CH)])
            return carry

        lax.fori_loop(0, nsub, ibody, 0)

    def accumulate(srcs):
        for ck, src_hbm in enumerate(srcs):
            cbase = wid * _EPWC

            def fire(i, p, src_hbm=src_hbm, ck=ck):
                pltpu.sync_copy(dst_hbm.at[pl.ds(ck * _EC + cbase + i * _CH, _CH)],
                                ix[p])
                pltpu.async_copy(src_hbm.at[pl.ds(cbase + i * _CH, _CH)],
                                 bb[p], sl[p])

            def wait_rows(i, p, src_hbm=src_hbm):
                pltpu.make_async_copy(src_hbm.at[pl.ds(cbase + i * _CH, _CH)],
                                      bb[p], sl[p]).wait()

            def add(p):
                pltpu.sync_copy(bb[p], acc_sh.at[ix[p]], add=True)

            fire(0, 0)

            def body(k, carry):
                i0 = 2 * k
                fire(i0 + 1, 1)
                wait_rows(i0, 0)
                add(0)

                @pl.when(k < _NPAIR - 1)
                def _():
                    fire(i0 + 2, 0)

                wait_rows(i0 + 1, 1)
                add(1)
                return carry

            lax.fori_loop(0, _NPAIR, body, 0)

    def dump(out_hbm):
        def obody(j, carry):
            pltpu.sync_copy(acc_sh.at[pl.ds(slab + j * _CH, _CH)], b0)
            pltpu.sync_copy(b0, out_hbm.at[pl.ds(obase + j * _CH, _CH)])
            return carry

        lax.fori_loop(0, nsub, obody, 0)

    zero_vmem()
    zero_slab()
    plsc.subcore_barrier()
    accumulate([p0, p1, p2, p3, p4])
    plsc.subcore_barrier()
    dump(macc_out)
    zero_vmem()
    zero_slab()
    plsc.subcore_barrier()
    accumulate([w0, w1, w2, w3, w4])
    plsc.subcore_barrier()
    dump(wacc_out)


def _node_body(x_ref, m_ref, wa_ref, wo_ref, nw1a_ref, nw1b_ref, nb1_ref,
               nw2_ref, nb2_ref, g_ref, b_ref, out_ref):
    xv = x_ref[...]
    macc = m_ref[0] + m_ref[1]
    wacc = wa_ref[0] + wa_ref[1]
    msg = macc / (wacc + 1e-12)
    msgo = jnp.dot(msg, wo_ref[...], preferred_element_type=_F32)
    h1 = (jnp.dot(xv, nw1a_ref[...], preferred_element_type=_F32)
          + jnp.dot(msgo, nw1b_ref[...], preferred_element_type=_F32)
          + nb1_ref[...])
    h1 = jnp.maximum(h1, 0.0)
    h2 = jnp.dot(h1, nw2_ref[...], preferred_element_type=_F32) + nb2_ref[...]
    mu = jnp.mean(h2, axis=-1, keepdims=True)
    dcen = h2 - mu
    var = jnp.mean(dcen * dcen, axis=-1, keepdims=True)
    out_ref[...] = xv + g_ref[...] * dcen / jnp.sqrt(var + 1e-5) + b_ref[...]


def _node(x, macc2, wacc2, wo, nw1a, nw1b, nb1, nw2, nb2, g, b):
    cmat = lambda shape: pl.BlockSpec(shape, lambda i: (0, 0))
    return pl.pallas_call(
        _node_body,
        grid=(_N // _BN,),
        in_specs=[
            pl.BlockSpec((_BN, _H), lambda i: (i, 0)),
            pl.BlockSpec((2, _BN, _H), lambda i: (0, i, 0)),
            pl.BlockSpec((2, _BN, _H), lambda i: (0, i, 0)),
            cmat((_H, _H)), cmat((_H, _H)), cmat((_H, _H)), cmat((1, _H)),
            cmat((_H, _H)), cmat((1, _H)), cmat((1, _H)), cmat((1, _H)),
        ],
        out_specs=pl.BlockSpec((_BN, _H), lambda i: (i, 0)),
        out_shape=jax.ShapeDtypeStruct((_N, _H), _F32),
    )(x, macc2, wacc2, wo, nw1a, nw1b, nb1, nw2, nb2, g, b)


def kernel(x, edge_index, edge_attr, edge_w1, edge_b1, edge_w2, edge_b2,
           edge_ln_g, edge_ln_b, node_w1, node_b1, node_w2, node_b2,
           node_ln_g, node_ln_b, wq, wk, wv, wo):
    src = edge_index[0].astype(jnp.int32)
    dst = edge_index[1].astype(jnp.int32)

    w1a = edge_w1[:_H]
    w1b = edge_w1[_H:2 * _H]
    w1c = edge_w1[2 * _H:]
    wd = jnp.concatenate([w1a, wq], axis=1)
    ws = jnp.concatenate([w1b, wk], axis=1)

    head_of = jnp.arange(_H, dtype=jnp.int32) // _DK
    lane = jnp.arange(16, dtype=jnp.int32)
    seg16 = (head_of[:, None] == lane[None, :]).astype(_F32) / math.sqrt(_DK)
    exp16 = (lane[:, None] == head_of[None, :]).astype(_F32)

    row = lambda v: v.reshape(1, _H)

    td, ts = _proj(x, wd, ws)

    e_cs, p_cs, w_cs = [], [], []
    for ck in range(_NCK):
        dst_c = lax.dynamic_slice_in_dim(dst, ck * _EC, _EC)
        src_c = lax.dynamic_slice_in_dim(src, ck * _EC, _EC)
        gd, gs = _sc_gather(td, ts, dst_c, src_c)
        e_c, p_c, w_c = _edge(gd, gs, edge_attr, w1c, row(edge_b1), edge_w2,
                              row(edge_b2), row(edge_ln_g), row(edge_ln_b),
                              wv, seg16, exp16, ck)
        e_cs.append(e_c)
        p_cs.append(p_c)
        w_cs.append(w_c)

    macc2, wacc2 = _sc_scatter(*p_cs, *w_cs, dst)
    x_new = _node(x, macc2.reshape(2, _NP, _H), wacc2.reshape(2, _NP, _H),
                  wo, node_w1[:_H], node_w1[_H:], row(node_b1), node_w2,
                  row(node_b2), row(node_ln_g), row(node_ln_b))
    return (x_new, jnp.concatenate(e_cs, axis=0))


# speedup vs baseline: 1.0587x; 1.0587x over previous
import functools
import math

import jax
import jax.numpy as jnp
from jax import lax
from jax.experimental import pallas as pl
from jax.experimental.pallas import tpu as pltpu
from jax.experimental.pallas import tpu_sc as plsc

_N = 10000
_E = 320000
_H = 128
_NH = 4
_DK = _H // _NH

_NC = 2
_NS = 16
_NW = _NC * _NS
_CH = 40
_NP = 10240
_RPS = _NP // _NS

_NCK = 5
_EC = _E // _NCK
_EPWC = _EC // _NW
_NSUBC = _EPWC // _CH
_NPAIR = _NSUBC // 2
_NSP = 56

_BE = 512
_EBLK = _EC // _BE
_BN = 1000

_F32 = jnp.float32


_U32 = jnp.uint32


def _pack16(lo, hi):
    lo_u = jax.lax.bitcast_convert_type(lo, _U32)
    hi_u = jax.lax.bitcast_convert_type(hi, _U32)
    word = (lo_u >> _U32(16)) | (hi_u & _U32(0xFFFF0000))
    return jax.lax.bitcast_convert_type(word, _F32)


def _unpack16(word):
    u = jax.lax.bitcast_convert_type(word, _U32)
    lo = jax.lax.bitcast_convert_type(u << _U32(16), _F32)
    hi = jax.lax.bitcast_convert_type(u & _U32(0xFFFF0000), _F32)
    return lo, hi


def _proj_body(x_ref, wd_ref, ws_ref, td_ref, ts_ref):
    xv = x_ref[...]
    a = jnp.dot(xv, wd_ref[:, :_H], preferred_element_type=_F32)
    q = jnp.dot(xv, wd_ref[:, _H:], preferred_element_type=_F32)
    b = jnp.dot(xv, ws_ref[:, :_H], preferred_element_type=_F32)
    k = jnp.dot(xv, ws_ref[:, _H:], preferred_element_type=_F32)
    td_ref[...] = _pack16(a, q)
    ts_ref[...] = _pack16(b, k)


def _proj(x, wd, ws):
    return pl.pallas_call(
        _proj_body,
        grid=(_N // _BN,),
        in_specs=[
            pl.BlockSpec((_BN, _H), lambda i: (i, 0)),
            pl.BlockSpec((_H, 2 * _H), lambda i: (0, 0)),
            pl.BlockSpec((_H, 2 * _H), lambda i: (0, 0)),
        ],
        out_specs=[
            pl.BlockSpec((_BN, _H), lambda i: (i, 0)),
            pl.BlockSpec((_BN, _H), lambda i: (i, 0)),
        ],
        out_shape=[
            jax.ShapeDtypeStruct((_N, _H), _F32),
            jax.ShapeDtypeStruct((_N, _H), _F32),
        ],
    )(x, wd, ws)


_sc_mesh = plsc.VectorSubcoreMesh(core_axis_name="c", subcore_axis_name="s")


@functools.partial(
    pl.kernel,
    mesh=_sc_mesh,
    out_type=[
        jax.ShapeDtypeStruct((_EC, _H), _F32),
        jax.ShapeDtypeStruct((_EC, _H), _F32),
    ],
    scratch_types=[
        pltpu.VMEM((_NSP, _CH), jnp.int32),
        pltpu.VMEM((_NSP, _CH), jnp.int32),
        pltpu.VMEM((_CH, _H), _F32),
        pltpu.VMEM((_CH, _H), _F32),
        pltpu.VMEM((_CH, _H), _F32),
        pltpu.VMEM((_CH, _H), _F32),
        pltpu.SemaphoreType.DMA,
        pltpu.SemaphoreType.DMA,
        pltpu.SemaphoreType.DMA,
        pltpu.SemaphoreType.DMA,
        pltpu.SemaphoreType.DMA,
        pltpu.SemaphoreType.DMA,
        pltpu.SemaphoreType.DMA,
        pltpu.SemaphoreType.DMA,
    ],
)
def _sc_gather(td_hbm, ts_hbm, dst2_hbm, src2_hbm, gd_out, gs_out,
               ixd_all, ixs_all, rd0, rd1, rs0, rs1,
               sgd0, sgd1, sgs0, sgs1, sod0, sod1, sos0, sos1):
    wid = lax.axis_index("s") * _NC + lax.axis_index("c")
    base0 = wid * _EPWC
    rrow0 = wid * _NSP
    rd = (rd0, rd1)
    rs = (rs0, rs1)
    sgd = (sgd0, sgd1)
    sgs = (sgs0, sgs1)
    sod = (sod0, sod1)
    sos = (sos0, sos1)

    pltpu.sync_copy(dst2_hbm.at[pl.ds(rrow0, _NSP)], ixd_all)
    pltpu.sync_copy(src2_hbm.at[pl.ds(rrow0, _NSP)], ixs_all)

    def fire(i, p):
        pltpu.async_copy(td_hbm.at[ixd_all.at[i]], rd[p], sgd[p])
        pltpu.async_copy(ts_hbm.at[ixs_all.at[i]], rs[p], sgs[p])

    def wait_gather(i, p):
        pltpu.make_async_copy(td_hbm.at[ixd_all.at[i]], rd[p], sgd[p]).wait()
        pltpu.make_async_copy(ts_hbm.at[ixs_all.at[i]], rs[p], sgs[p]).wait()

    def store(i, p):
        b = base0 + i * _CH
        pltpu.async_copy(rd[p], gd_out.at[pl.ds(b, _CH)], sod[p])
        pltpu.async_copy(rs[p], gs_out.at[pl.ds(b, _CH)], sos[p])

    def wait_store(i, p):
        b = base0 + i * _CH
        pltpu.make_async_copy(rd[p], gd_out.at[pl.ds(b, _CH)], sod[p]).wait()
        pltpu.make_async_copy(rs[p], gs_out.at[pl.ds(b, _CH)], sos[p]).wait()

    fire(0, 0)

    def body(k, carry):
        i0 = 2 * k
        fire(i0 + 1, 1)
        wait_gather(i0, 0)
        store(i0, 0)
        wait_store(i0, 0)

        @pl.when(k < _NPAIR - 1)
        def _():
            fire(i0 + 2, 0)

        wait_gather(i0 + 1, 1)
        store(i0 + 1, 1)
        wait_store(i0 + 1, 1)
        return carry

    lax.fori_loop(0, _NPAIR, body, 0)


def _edge_body(gd_ref, gs_ref, ea_ref, w1c_ref, b1_ref, w2_ref, b2_ref,
               g_ref, b_ref, wv_ref, seg_ref, exp_ref,
               eout_ref, pout_ref, wout_ref):
    bf = jnp.bfloat16
    ea = ea_ref[...]
    ad, qd = _unpack16(gd_ref[...])
    bs, ks = _unpack16(gs_ref[...])
    h1 = jnp.dot(ea.astype(bf), w1c_ref[...].astype(bf),
                 preferred_element_type=_F32)
    h1 = jnp.maximum(h1 + ad + bs + b1_ref[...], 0.0)
    h2 = jnp.dot(h1.astype(bf), w2_ref[...].astype(bf),
                 preferred_element_type=_F32) + b2_ref[...]
    mu = jnp.mean(h2, axis=-1, keepdims=True)
    dcen = h2 - mu
    var = jnp.mean(dcen * dcen, axis=-1, keepdims=True)
    e_new = g_ref[...] * dcen / jnp.sqrt(var + 1e-5) + b_ref[...]
    eout_ref[...] = ea + e_new
    prod = qd * ks
    w16 = jnp.exp(jnp.dot(prod.astype(bf), seg_ref[...].astype(bf),
                          preferred_element_type=_F32))
    wb = jnp.dot(w16.astype(bf), exp_ref[...].astype(bf),
                 preferred_element_type=_F32)
    wout_ref[...] = wb
    v = jnp.dot(e_new.astype(bf), wv_ref[...].astype(bf),
                preferred_element_type=_F32)
    pout_ref[...] = v * wb


def _edge(gd, gs, ea, w1c, b1, w2, b2, g, b, wv, seg16, exp16, ck):
    cmat = lambda shape: pl.BlockSpec(shape, lambda i: (0, 0))
    return pl.pallas_call(
        _edge_body,
        grid=(_EBLK,),
        in_specs=[
            pl.BlockSpec((_BE, _H), lambda i: (i, 0)),
            pl.BlockSpec((_BE, _H), lambda i: (i, 0)),
            pl.BlockSpec((_BE, _H), lambda i, c=ck: (i + c * _EBLK, 0)),
            cmat((_H, _H)), cmat((1, _H)), cmat((_H, _H)), cmat((1, _H)),
            cmat((1, _H)), cmat((1, _H)), cmat((_H, _H)),
            cmat((_H, 16)), cmat((16, _H)),
        ],
        out_specs=[
            pl.BlockSpec((_BE, _H), lambda i: (i, 0)),
            pl.BlockSpec((_BE, _H), lambda i: (i, 0)),
            pl.BlockSpec((_BE, _H), lambda i: (i, 0)),
        ],
        out_shape=[
            jax.ShapeDtypeStruct((_EC, _H), _F32),
            jax.ShapeDtypeStruct((_EC, _H), _F32),
            jax.ShapeDtypeStruct((_EC, _H), _F32),
        ],
    )(gd, gs, ea, w1c, b1, w2, b2, g, b, wv, seg16, exp16)


@functools.partial(
    pl.kernel,
    mesh=_sc_mesh,
    out_type=[
        jax.ShapeDtypeStruct((_NC * _NP, _H), _F32),
        jax.ShapeDtypeStruct((_NC * _NP, _H), _F32),
    ],
    scratch_types=[
        pltpu.VMEM((_NSP, _CH), jnp.int32),
        pltpu.VMEM((_CH, _H), _F32),
        pltpu.VMEM((_CH, _H), _F32),
        pltpu.SemaphoreType.DMA,
        pltpu.SemaphoreType.DMA,
        pltpu.VMEM_SHARED((_NP, _H), _F32),
    ],
)
def _sc_scatter(p0, p1, p2, p3, p4, w0, w1, w2, w3, w4, dst2_hbm,
                macc_out, wacc_out, ixb, b0, b1, sl0, sl1, acc_sh):
    cid = lax.axis_index("c")
    sid = lax.axis_index("s")
    wid = sid * _NC + cid
    slab = sid * _RPS
    nsub = _RPS // _CH
    obase = cid * _NP + slab
    zv = jnp.zeros((16,), _F32)
    bb = (b0, b1)
    sl = (sl0, sl1)

    def zero_vmem():
        def zbody(r, carry):
            for cc in range(_H // 16):
                b0[r, pl.ds(cc * 16, 16)] = zv
            return carry

        lax.fori_loop(0, _CH, zbody, 0)

    def zero_slab():
        def ibody(j, carry):
            pltpu.sync_copy(b0, acc_sh.at[pl.ds(slab + j * _CH, _CH)])
            return carry

        lax.fori_loop(0, nsub, ibody, 0)

    def accumulate(srcs):
        for ck, src_hbm in enumerate(srcs):
            cbase = wid * _EPWC
            pltpu.sync_copy(
                dst2_hbm.at[pl.ds((ck * _NW + wid) * _NSP, _NSP)], ixb)

            def fire(i, p, src_hbm=src_hbm):
                pltpu.async_copy(src_hbm.at[pl.ds(cbase + i * _CH, _CH)],
                                 bb[p], sl[p])

            def wait_rows(i, p, src_hbm=src_hbm):
                pltpu.make_async_copy(src_hbm.at[pl.ds(cbase + i * _CH, _CH)],
                                      bb[p], sl[p]).wait()

            def add(i, p):
                pltpu.sync_copy(bb[p], acc_sh.at[ixb.at[i]], add=True)

            fire(0, 0)

            def body(k, carry):
                i0 = 2 * k
                fire(i0 + 1, 1)
                wait_rows(i0, 0)
                add(i0, 0)

                @pl.when(k < _NPAIR - 1)
                def _():
                    fire(i0 + 2, 0)

                wait_rows(i0 + 1, 1)
                add(i0 + 1, 1)
                return carry

            lax.fori_loop(0, _NPAIR, body, 0)

    def dump(out_hbm):
        def obody(j, carry):
            pltpu.sync_copy(acc_sh.at[pl.ds(slab + j * _CH, _CH)], b0)
            pltpu.sync_copy(b0, out_hbm.at[pl.ds(obase + j * _CH, _CH)])
            return carry

        lax.fori_loop(0, nsub, obody, 0)

    zero_vmem()
    zero_slab()
    plsc.subcore_barrier()
    accumulate([p0, p1, p2, p3, p4])
    plsc.subcore_barrier()
    dump(macc_out)
    zero_vmem()
    zero_slab()
    plsc.subcore_barrier()
    accumulate([w0, w1, w2, w3, w4])
    plsc.subcore_barrier()
    dump(wacc_out)


def _node_body(x_ref, m_ref, wa_ref, wo_ref, nw1a_ref, nw1b_ref, nb1_ref,
               nw2_ref, nb2_ref, g_ref, b_ref, out_ref):
    xv = x_ref[...]
    macc = m_ref[0] + m_ref[1]
    wacc = wa_ref[0] + wa_ref[1]
    msg = macc / (wacc + 1e-12)
    msgo = jnp.dot(msg, wo_ref[...], preferred_element_type=_F32)
    h1 = (jnp.dot(xv, nw1a_ref[...], preferred_element_type=_F32)
          + jnp.dot(msgo, nw1b_ref[...], preferred_element_type=_F32)
          + nb1_ref[...])
    h1 = jnp.maximum(h1, 0.0)
    h2 = jnp.dot(h1, nw2_ref[...], preferred_element_type=_F32) + nb2_ref[...]
    mu = jnp.mean(h2, axis=-1, keepdims=True)
    dcen = h2 - mu
    var = jnp.mean(dcen * dcen, axis=-1, keepdims=True)
    out_ref[...] = xv + g_ref[...] * dcen / jnp.sqrt(var + 1e-5) + b_ref[...]


def _node(x, macc2, wacc2, wo, nw1a, nw1b, nb1, nw2, nb2, g, b):
    cmat = lambda shape: pl.BlockSpec(shape, lambda i: (0, 0))
    return pl.pallas_call(
        _node_body,
        grid=(_N // _BN,),
        in_specs=[
            pl.BlockSpec((_BN, _H), lambda i: (i, 0)),
            pl.BlockSpec((2, _BN, _H), lambda i: (0, i, 0)),
            pl.BlockSpec((2, _BN, _H), lambda i: (0, i, 0)),
            cmat((_H, _H)), cmat((_H, _H)), cmat((_H, _H)), cmat((1, _H)),
            cmat((_H, _H)), cmat((1, _H)), cmat((1, _H)), cmat((1, _H)),
        ],
        out_specs=pl.BlockSpec((_BN, _H), lambda i: (i, 0)),
        out_shape=jax.ShapeDtypeStruct((_N, _H), _F32),
    )(x, macc2, wacc2, wo, nw1a, nw1b, nb1, nw2, nb2, g, b)


def kernel(x, edge_index, edge_attr, edge_w1, edge_b1, edge_w2, edge_b2,
           edge_ln_g, edge_ln_b, node_w1, node_b1, node_w2, node_b2,
           node_ln_g, node_ln_b, wq, wk, wv, wo):
    src = edge_index[0].astype(jnp.int32)
    dst = edge_index[1].astype(jnp.int32)

    w1a = edge_w1[:_H]
    w1b = edge_w1[_H:2 * _H]
    w1c = edge_w1[2 * _H:]
    wd = jnp.concatenate([w1a, wq], axis=1)
    ws = jnp.concatenate([w1b, wk], axis=1)

    head_of = jnp.arange(_H, dtype=jnp.int32) // _DK
    lane = jnp.arange(16, dtype=jnp.int32)
    seg16 = (head_of[:, None] == lane[None, :]).astype(_F32) / math.sqrt(_DK)
    exp16 = (lane[:, None] == head_of[None, :]).astype(_F32)

    row = lambda v: v.reshape(1, _H)

    td, ts = _proj(x, wd, ws)

    def padded_idx(v):
        v4 = v.reshape(_NCK, _NW, _NSUBC, _CH)
        v4 = jnp.pad(v4, ((0, 0), (0, 0), (0, _NSP - _NSUBC), (0, 0)))
        return v4.reshape(_NCK, _NW * _NSP, _CH)

    dst2 = padded_idx(dst)
    src2 = padded_idx(src)

    e_cs, p_cs, w_cs = [], [], []
    for ck in range(_NCK):
        gd, gs = _sc_gather(td, ts, dst2[ck], src2[ck])
        e_c, p_c, w_c = _edge(gd, gs, edge_attr, w1c, row(edge_b1), edge_w2,
                              row(edge_b2), row(edge_ln_g), row(edge_ln_b),
                              wv, seg16, exp16, ck)
        e_cs.append(e_c)
        p_cs.append(p_c)
        w_cs.append(w_c)

    macc2, wacc2 = _sc_scatter(*p_cs, *w_cs,
                               dst2.reshape(_NCK * _NW * _NSP, _CH))
    x_new = _node(x, macc2.reshape(2, _NP, _H), wacc2.reshape(2, _NP, _H),
                  wo, node_w1[:_H], node_w1[_H:], row(node_b1), node_w2,
                  row(node_b2), row(node_ln_g), row(node_ln_b))
    return (x_new, jnp.concatenate(e_cs, axis=0))


# speedup vs baseline: 1.1450x; 1.0815x over previous
import functools
import math

import jax
import jax.numpy as jnp
from jax import lax
from jax.experimental import pallas as pl
from jax.experimental.pallas import tpu as pltpu
from jax.experimental.pallas import tpu_sc as plsc

_N = 10000
_E = 320000
_H = 128
_NH = 4
_DK = _H // _NH

_NC = 2
_NS = 16
_NW = _NC * _NS
_CH = 40
_NP = 10240
_RPS = _NP // _NS

_NCK = 5
_EC = _E // _NCK
_EPWC = _EC // _NW
_NSUBC = _EPWC // _CH
_NPAIR = _NSUBC // 2
_NSP = 56

_BE = 512
_EBLK = _EC // _BE
_BN = 1000

_F32 = jnp.float32


_U32 = jnp.uint32


def _pack16(lo, hi):
    lo_u = jax.lax.bitcast_convert_type(lo, _U32)
    hi_u = jax.lax.bitcast_convert_type(hi, _U32)
    word = (lo_u >> _U32(16)) | (hi_u & _U32(0xFFFF0000))
    return jax.lax.bitcast_convert_type(word, _F32)


def _unpack16(word):
    u = jax.lax.bitcast_convert_type(word, _U32)
    lo = jax.lax.bitcast_convert_type(u << _U32(16), _F32)
    hi = jax.lax.bitcast_convert_type(u & _U32(0xFFFF0000), _F32)
    return lo, hi


def _proj_body(x_ref, wd_ref, ws_ref, td_ref, ts_ref):
    xv = x_ref[...]
    a = jnp.dot(xv, wd_ref[:, :_H], preferred_element_type=_F32)
    q = jnp.dot(xv, wd_ref[:, _H:], preferred_element_type=_F32)
    b = jnp.dot(xv, ws_ref[:, :_H], preferred_element_type=_F32)
    k = jnp.dot(xv, ws_ref[:, _H:], preferred_element_type=_F32)
    td_ref[...] = _pack16(a, q)
    ts_ref[...] = _pack16(b, k)


def _proj(x, wd, ws):
    return pl.pallas_call(
        _proj_body,
        grid=(_N // _BN,),
        in_specs=[
            pl.BlockSpec((_BN, _H), lambda i: (i, 0)),
            pl.BlockSpec((_H, 2 * _H), lambda i: (0, 0)),
            pl.BlockSpec((_H, 2 * _H), lambda i: (0, 0)),
        ],
        out_specs=[
            pl.BlockSpec((_BN, _H), lambda i: (i, 0)),
            pl.BlockSpec((_BN, _H), lambda i: (i, 0)),
        ],
        out_shape=[
            jax.ShapeDtypeStruct((_N, _H), _F32),
            jax.ShapeDtypeStruct((_N, _H), _F32),
        ],
    )(x, wd, ws)


_sc_mesh = plsc.VectorSubcoreMesh(core_axis_name="c", subcore_axis_name="s")


@functools.partial(
    pl.kernel,
    mesh=_sc_mesh,
    out_type=[
        jax.ShapeDtypeStruct((_EC, _H), _F32),
        jax.ShapeDtypeStruct((_EC, _H), _F32),
    ],
    scratch_types=[
        pltpu.VMEM((_NSP, _CH), jnp.int32),
        pltpu.VMEM((_NSP, _CH), jnp.int32),
        pltpu.VMEM((_CH, _H), _F32),
        pltpu.VMEM((_CH, _H), _F32),
        pltpu.VMEM((_CH, _H), _F32),
        pltpu.VMEM((_CH, _H), _F32),
        pltpu.SemaphoreType.DMA,
        pltpu.SemaphoreType.DMA,
        pltpu.SemaphoreType.DMA,
        pltpu.SemaphoreType.DMA,
        pltpu.SemaphoreType.DMA,
        pltpu.SemaphoreType.DMA,
        pltpu.SemaphoreType.DMA,
        pltpu.SemaphoreType.DMA,
    ],
)
def _sc_gather(td_hbm, ts_hbm, dst2_hbm, src2_hbm, gd_out, gs_out,
               ixd_all, ixs_all, rd0, rd1, rs0, rs1,
               sgd0, sgd1, sgs0, sgs1, sod0, sod1, sos0, sos1):
    wid = lax.axis_index("s") * _NC + lax.axis_index("c")
    base0 = wid * _EPWC
    rrow0 = wid * _NSP
    rd = (rd0, rd1)
    rs = (rs0, rs1)
    sgd = (sgd0, sgd1)
    sgs = (sgs0, sgs1)
    sod = (sod0, sod1)
    sos = (sos0, sos1)

    pltpu.sync_copy(dst2_hbm.at[pl.ds(rrow0, _NSP)], ixd_all)
    pltpu.sync_copy(src2_hbm.at[pl.ds(rrow0, _NSP)], ixs_all)

    def fire(i, p):
        pltpu.async_copy(td_hbm.at[ixd_all.at[i]], rd[p], sgd[p])
        pltpu.async_copy(ts_hbm.at[ixs_all.at[i]], rs[p], sgs[p])

    def wait_gather(i, p):
        pltpu.make_async_copy(td_hbm.at[ixd_all.at[i]], rd[p], sgd[p]).wait()
        pltpu.make_async_copy(ts_hbm.at[ixs_all.at[i]], rs[p], sgs[p]).wait()

    def store(i, p):
        b = base0 + i * _CH
        pltpu.async_copy(rd[p], gd_out.at[pl.ds(b, _CH)], sod[p])
        pltpu.async_copy(rs[p], gs_out.at[pl.ds(b, _CH)], sos[p])

    def wait_store(i, p):
        b = base0 + i * _CH
        pltpu.make_async_copy(rd[p], gd_out.at[pl.ds(b, _CH)], sod[p]).wait()
        pltpu.make_async_copy(rs[p], gs_out.at[pl.ds(b, _CH)], sos[p]).wait()

    fire(0, 0)

    def body(k, carry):
        i0 = 2 * k
        fire(i0 + 1, 1)
        wait_gather(i0, 0)
        store(i0, 0)
        wait_store(i0, 0)

        @pl.when(k < _NPAIR - 1)
        def _():
            fire(i0 + 2, 0)

        wait_gather(i0 + 1, 1)
        store(i0 + 1, 1)
        wait_store(i0 + 1, 1)
        return carry

    lax.fori_loop(0, _NPAIR, body, 0)


def _edge_body(gd_ref, gs_ref, ea_ref, w1c_ref, b1_ref, w2_ref, b2_ref,
               g_ref, b_ref, wv_ref, seg_ref, exp_ref,
               eout_ref, pout_ref, wout_ref):
    bf = jnp.bfloat16
    ea = ea_ref[...]
    ad, qd = _unpack16(gd_ref[...])
    bs, ks = _unpack16(gs_ref[...])
    h1 = jnp.dot(ea.astype(bf), w1c_ref[...].astype(bf),
                 preferred_element_type=_F32)
    h1 = jnp.maximum(h1 + ad + bs + b1_ref[...], 0.0)
    h2 = jnp.dot(h1.astype(bf), w2_ref[...].astype(bf),
                 preferred_element_type=_F32) + b2_ref[...]
    mu = jnp.mean(h2, axis=-1, keepdims=True)
    dcen = h2 - mu
    var = jnp.mean(dcen * dcen, axis=-1, keepdims=True)
    e_new = g_ref[...] * dcen / jnp.sqrt(var + 1e-5) + b_ref[...]
    eout_ref[...] = ea + e_new
    prod = qd * ks
    w16 = jnp.exp(jnp.dot(prod.astype(bf), seg_ref[...].astype(bf),
                          preferred_element_type=_F32))
    wb = jnp.dot(w16.astype(bf), exp_ref[...].astype(bf),
                 preferred_element_type=_F32)
    wout_ref[...] = wb
    v = jnp.dot(e_new.astype(bf), wv_ref[...].astype(bf),
                preferred_element_type=_F32)
    pout_ref[...] = v * wb


def _edge(gd, gs, ea, w1c, b1, w2, b2, g, b, wv, seg16, exp16, ck):
    cmat = lambda shape: pl.BlockSpec(shape, lambda i: (0, 0))
    return pl.pallas_call(
        _edge_body,
        grid=(_EBLK,),
        in_specs=[
            pl.BlockSpec((_BE, _H), lambda i: (i, 0)),
            pl.BlockSpec((_BE, _H), lambda i: (i, 0)),
            pl.BlockSpec((_BE, _H), lambda i, c=ck: (i + c * _EBLK, 0)),
            cmat((_H, _H)), cmat((1, _H)), cmat((_H, _H)), cmat((1, _H)),
            cmat((1, _H)), cmat((1, _H)), cmat((_H, _H)),
            cmat((_H, 16)), cmat((16, _H)),
        ],
        out_specs=[
            pl.BlockSpec((_BE, _H), lambda i: (i, 0)),
            pl.BlockSpec((_BE, _H), lambda i: (i, 0)),
            pl.BlockSpec((_BE, _H), lambda i: (i, 0)),
        ],
        out_shape=[
            jax.ShapeDtypeStruct((_EC, _H), _F32),
            jax.ShapeDtypeStruct((_EC, _H), _F32),
            jax.ShapeDtypeStruct((_EC, _H), _F32),
        ],
    )(gd, gs, ea, w1c, b1, w2, b2, g, b, wv, seg16, exp16)


def _make_scatter(cks):
    ncs = len(cks)

    @functools.partial(
        pl.kernel,
        mesh=_sc_mesh,
        out_type=[
            jax.ShapeDtypeStruct((_NC * _NP, _H), _F32),
            jax.ShapeDtypeStruct((_NC * _NP, _H), _F32),
        ],
        scratch_types=[
            pltpu.VMEM((_NSP, _CH), jnp.int32),
            pltpu.VMEM((_CH, _H), _F32),
            pltpu.VMEM((_CH, _H), _F32),
            pltpu.SemaphoreType.DMA,
            pltpu.SemaphoreType.DMA,
            pltpu.VMEM_SHARED((_NP, _H), _F32),
        ],
    )
    def scatter(*args):
        ps = args[:ncs]
        ws = args[ncs:2 * ncs]
        dst2_hbm = args[2 * ncs]
        macc_out, wacc_out, ixb, b0, b1, sl0, sl1, acc_sh = args[2 * ncs + 1:]
        cid = lax.axis_index("c")
        sid = lax.axis_index("s")
        wid = sid * _NC + cid
        slab = sid * _RPS
        nsub = _RPS // _CH
        obase = cid * _NP + slab
        zv = jnp.zeros((16,), _F32)
        bb = (b0, b1)
        sl = (sl0, sl1)

        def zero_vmem():
            def zbody(r, carry):
                for cc in range(_H // 16):
                    b0[r, pl.ds(cc * 16, 16)] = zv
                return carry

            lax.fori_loop(0, _CH, zbody, 0)

        def zero_slab():
            def ibody(j, carry):
                pltpu.sync_copy(b0, acc_sh.at[pl.ds(slab + j * _CH, _CH)])
                return carry

            lax.fori_loop(0, nsub, ibody, 0)

        def accumulate(srcs):
            for ck, src_hbm in zip(cks, srcs):
                cbase = wid * _EPWC
                pltpu.sync_copy(
                    dst2_hbm.at[pl.ds((ck * _NW + wid) * _NSP, _NSP)], ixb)

                def fire(i, p, src_hbm=src_hbm):
                    pltpu.async_copy(src_hbm.at[pl.ds(cbase + i * _CH, _CH)],
                                     bb[p], sl[p])

                def wait_rows(i, p, src_hbm=src_hbm):
                    pltpu.make_async_copy(
                        src_hbm.at[pl.ds(cbase + i * _CH, _CH)],
                        bb[p], sl[p]).wait()

                def add(i, p):
                    pltpu.sync_copy(bb[p], acc_sh.at[ixb.at[i]], add=True)

                fire(0, 0)

                def body(k, carry):
                    i0 = 2 * k
                    fire(i0 + 1, 1)
                    wait_rows(i0, 0)
                    add(i0, 0)

                    @pl.when(k < _NPAIR - 1)
                    def _():
                        fire(i0 + 2, 0)

                    wait_rows(i0 + 1, 1)
                    add(i0 + 1, 1)
                    return carry

                lax.fori_loop(0, _NPAIR, body, 0)

        def dump(out_hbm):
            def obody(j, carry):
                pltpu.sync_copy(acc_sh.at[pl.ds(slab + j * _CH, _CH)], b0)
                pltpu.sync_copy(b0, out_hbm.at[pl.ds(obase + j * _CH, _CH)])
                return carry

            lax.fori_loop(0, nsub, obody, 0)

        zero_vmem()
        zero_slab()
        plsc.subcore_barrier()
        accumulate(ps)
        plsc.subcore_barrier()
        dump(macc_out)
        zero_vmem()
        zero_slab()
        plsc.subcore_barrier()
        accumulate(ws)
        plsc.subcore_barrier()
        dump(wacc_out)

    return scatter


_sc_scatter_a = _make_scatter((0, 1, 2))
_sc_scatter_b = _make_scatter((3, 4))


def _node_body(x_ref, m_ref, wa_ref, mb_ref, wb_ref, wo_ref, nw1a_ref,
               nw1b_ref, nb1_ref, nw2_ref, nb2_ref, g_ref, b_ref, out_ref):
    xv = x_ref[...]
    macc = m_ref[0] + m_ref[1] + mb_ref[0] + mb_ref[1]
    wacc = wa_ref[0] + wa_ref[1] + wb_ref[0] + wb_ref[1]
    msg = macc / (wacc + 1e-12)
    msgo = jnp.dot(msg, wo_ref[...], preferred_element_type=_F32)
    h1 = (jnp.dot(xv, nw1a_ref[...], preferred_element_type=_F32)
          + jnp.dot(msgo, nw1b_ref[...], preferred_element_type=_F32)
          + nb1_ref[...])
    h1 = jnp.maximum(h1, 0.0)
    h2 = jnp.dot(h1, nw2_ref[...], preferred_element_type=_F32) + nb2_ref[...]
    mu = jnp.mean(h2, axis=-1, keepdims=True)
    dcen = h2 - mu
    var = jnp.mean(dcen * dcen, axis=-1, keepdims=True)
    out_ref[...] = xv + g_ref[...] * dcen / jnp.sqrt(var + 1e-5) + b_ref[...]


def _node(x, macc2, wacc2, macc2b, wacc2b, wo, nw1a, nw1b, nb1, nw2, nb2, g, b):
    cmat = lambda shape: pl.BlockSpec(shape, lambda i: (0, 0))
    return pl.pallas_call(
        _node_body,
        grid=(_N // _BN,),
        in_specs=[
            pl.BlockSpec((_BN, _H), lambda i: (i, 0)),
            pl.BlockSpec((2, _BN, _H), lambda i: (0, i, 0)),
            pl.BlockSpec((2, _BN, _H), lambda i: (0, i, 0)),
            pl.BlockSpec((2, _BN, _H), lambda i: (0, i, 0)),
            pl.BlockSpec((2, _BN, _H), lambda i: (0, i, 0)),
            cmat((_H, _H)), cmat((_H, _H)), cmat((_H, _H)), cmat((1, _H)),
            cmat((_H, _H)), cmat((1, _H)), cmat((1, _H)), cmat((1, _H)),
        ],
        out_specs=pl.BlockSpec((_BN, _H), lambda i: (i, 0)),
        out_shape=jax.ShapeDtypeStruct((_N, _H), _F32),
    )(x, macc2, wacc2, macc2b, wacc2b, wo, nw1a, nw1b, nb1, nw2, nb2, g, b)


def kernel(x, edge_index, edge_attr, edge_w1, edge_b1, edge_w2, edge_b2,
           edge_ln_g, edge_ln_b, node_w1, node_b1, node_w2, node_b2,
           node_ln_g, node_ln_b, wq, wk, wv, wo):
    src = edge_index[0].astype(jnp.int32)
    dst = edge_index[1].astype(jnp.int32)

    w1a = edge_w1[:_H]
    w1b = edge_w1[_H:2 * _H]
    w1c = edge_w1[2 * _H:]
    wd = jnp.concatenate([w1a, wq], axis=1)
    ws = jnp.concatenate([w1b, wk], axis=1)

    head_of = jnp.arange(_H, dtype=jnp.int32) // _DK
    lane = jnp.arange(16, dtype=jnp.int32)
    seg16 = (head_of[:, None] == lane[None, :]).astype(_F32) / math.sqrt(_DK)
    exp16 = (lane[:, None] == head_of[None, :]).astype(_F32)

    row = lambda v: v.reshape(1, _H)

    td, ts = _proj(x, wd, ws)

    def padded_idx(v):
        v4 = v.reshape(_NCK, _NW, _NSUBC, _CH)
        v4 = jnp.pad(v4, ((0, 0), (0, 0), (0, _NSP - _NSUBC), (0, 0)))
        return v4.reshape(_NCK, _NW * _NSP, _CH)

    dst2 = padded_idx(dst)
    src2 = padded_idx(src)

    e_cs, p_cs, w_cs = [], [], []
    for ck in range(_NCK):
        gd, gs = _sc_gather(td, ts, dst2[ck], src2[ck])
        e_c, p_c, w_c = _edge(gd, gs, edge_attr, w1c, row(edge_b1), edge_w2,
                              row(edge_b2), row(edge_ln_g), row(edge_ln_b),
                              wv, seg16, exp16, ck)
        e_cs.append(e_c)
        p_cs.append(p_c)
        w_cs.append(w_c)

    dst2f = dst2.reshape(_NCK * _NW * _NSP, _CH)
    macc_a, wacc_a = _sc_scatter_a(p_cs[0], p_cs[1], p_cs[2],
                                   w_cs[0], w_cs[1], w_cs[2], dst2f)
    macc_b, wacc_b = _sc_scatter_b(p_cs[3], p_cs[4], w_cs[3], w_cs[4], dst2f)
    x_new = _node(x, macc_a.reshape(2, _NP, _H), wacc_a.reshape(2, _NP, _H),
                  macc_b.reshape(2, _NP, _H), wacc_b.reshape(2, _NP, _H),
                  wo, node_w1[:_H], node_w1[_H:], row(node_b1), node_w2,
                  row(node_b2), row(node_ln_g), row(node_ln_b))
    return (x_new, jnp.concatenate(e_cs, axis=0))


# speedup vs baseline: 1.2677x; 1.1072x over previous
import functools
import math

import jax
import jax.numpy as jnp
from jax import lax
from jax.experimental import pallas as pl
from jax.experimental.pallas import tpu as pltpu
from jax.experimental.pallas import tpu_sc as plsc

_N = 10000
_E = 320000
_H = 128
_NH = 4
_DK = _H // _NH

_NC = 2
_NS = 16
_NW = _NC * _NS
_CH = 40
_NP = 10240
_RPS = _NP // _NS

_NCK = 5
_EC = _E // _NCK
_EPWC = _EC // _NW
_NSUBC = _EPWC // _CH
_NPAIR = _NSUBC // 2
_NSP = 56

_BE = 512
_EBLK = _EC // _BE
_BN = 1000

_F32 = jnp.float32


_U32 = jnp.uint32


def _pack16(lo, hi):
    lo_u = jax.lax.bitcast_convert_type(lo, _U32)
    hi_u = jax.lax.bitcast_convert_type(hi, _U32)
    word = (lo_u >> _U32(16)) | (hi_u & _U32(0xFFFF0000))
    return jax.lax.bitcast_convert_type(word, _F32)


def _unpack16(word):
    u = jax.lax.bitcast_convert_type(word, _U32)
    lo = jax.lax.bitcast_convert_type(u << _U32(16), _F32)
    hi = jax.lax.bitcast_convert_type(u & _U32(0xFFFF0000), _F32)
    return lo, hi


def _proj_body(x_ref, wd_ref, ws_ref, td_ref, ts_ref):
    xv = x_ref[...]
    a = jnp.dot(xv, wd_ref[:, :_H], preferred_element_type=_F32)
    q = jnp.dot(xv, wd_ref[:, _H:], preferred_element_type=_F32)
    b = jnp.dot(xv, ws_ref[:, :_H], preferred_element_type=_F32)
    k = jnp.dot(xv, ws_ref[:, _H:], preferred_element_type=_F32)
    td_ref[...] = _pack16(a, q)
    ts_ref[...] = _pack16(b, k)


def _proj(x, wd, ws):
    return pl.pallas_call(
        _proj_body,
        grid=(_N // _BN,),
        in_specs=[
            pl.BlockSpec((_BN, _H), lambda i: (i, 0)),
            pl.BlockSpec((_H, 2 * _H), lambda i: (0, 0)),
            pl.BlockSpec((_H, 2 * _H), lambda i: (0, 0)),
        ],
        out_specs=[
            pl.BlockSpec((_BN, _H), lambda i: (i, 0)),
            pl.BlockSpec((_BN, _H), lambda i: (i, 0)),
        ],
        out_shape=[
            jax.ShapeDtypeStruct((_N, _H), _F32),
            jax.ShapeDtypeStruct((_N, _H), _F32),
        ],
    )(x, wd, ws)


_sc_mesh = plsc.VectorSubcoreMesh(core_axis_name="c", subcore_axis_name="s")


@functools.partial(
    pl.kernel,
    mesh=_sc_mesh,
    out_type=[
        jax.ShapeDtypeStruct((_EC, _H), _F32),
        jax.ShapeDtypeStruct((_EC, _H), _F32),
    ],
    scratch_types=[
        pltpu.VMEM((_NSP, _CH), jnp.int32),
        pltpu.VMEM((_NSP, _CH), jnp.int32),
        pltpu.VMEM((_CH, _H), _F32),
        pltpu.VMEM((_CH, _H), _F32),
        pltpu.VMEM((_CH, _H), _F32),
        pltpu.VMEM((_CH, _H), _F32),
        pltpu.SemaphoreType.DMA,
        pltpu.SemaphoreType.DMA,
        pltpu.SemaphoreType.DMA,
        pltpu.SemaphoreType.DMA,
        pltpu.SemaphoreType.DMA,
        pltpu.SemaphoreType.DMA,
        pltpu.SemaphoreType.DMA,
        pltpu.SemaphoreType.DMA,
    ],
)
def _sc_gather(td_hbm, ts_hbm, dst2_hbm, src2_hbm, gd_out, gs_out,
               ixd_all, ixs_all, rd0, rd1, rs0, rs1,
               sgd0, sgd1, sgs0, sgs1, sod0, sod1, sos0, sos1):
    wid = lax.axis_index("s") * _NC + lax.axis_index("c")
    base0 = wid * _EPWC
    rrow0 = wid * _NSP
    rd = (rd0, rd1)
    rs = (rs0, rs1)
    sgd = (sgd0, sgd1)
    sgs = (sgs0, sgs1)
    sod = (sod0, sod1)
    sos = (sos0, sos1)

    pltpu.sync_copy(dst2_hbm.at[pl.ds(rrow0, _NSP)], ixd_all)
    pltpu.sync_copy(src2_hbm.at[pl.ds(rrow0, _NSP)], ixs_all)

    def fire(i, p):
        pltpu.async_copy(td_hbm.at[ixd_all.at[i]], rd[p], sgd[p])
        pltpu.async_copy(ts_hbm.at[ixs_all.at[i]], rs[p], sgs[p])

    def wait_gather(i, p):
        pltpu.make_async_copy(td_hbm.at[ixd_all.at[i]], rd[p], sgd[p]).wait()
        pltpu.make_async_copy(ts_hbm.at[ixs_all.at[i]], rs[p], sgs[p]).wait()

    def store(i, p):
        b = base0 + i * _CH
        pltpu.async_copy(rd[p], gd_out.at[pl.ds(b, _CH)], sod[p])
        pltpu.async_copy(rs[p], gs_out.at[pl.ds(b, _CH)], sos[p])

    def wait_store(i, p):
        b = base0 + i * _CH
        pltpu.make_async_copy(rd[p], gd_out.at[pl.ds(b, _CH)], sod[p]).wait()
        pltpu.make_async_copy(rs[p], gs_out.at[pl.ds(b, _CH)], sos[p]).wait()

    fire(0, 0)

    def body(k, carry):
        i0 = 2 * k
        fire(i0 + 1, 1)
        wait_gather(i0, 0)
        store(i0, 0)
        wait_store(i0, 0)

        @pl.when(k < _NPAIR - 1)
        def _():
            fire(i0 + 2, 0)

        wait_gather(i0 + 1, 1)
        store(i0 + 1, 1)
        wait_store(i0 + 1, 1)
        return carry

    lax.fori_loop(0, _NPAIR, body, 0)


def _edge_body(gd_ref, gs_ref, ea_ref, w1c_ref, b1_ref, w2_ref, b2_ref,
               g_ref, b_ref, wv_ref, seg_ref, exp_ref,
               eout_ref, pout_ref, wout_ref):
    bf = jnp.bfloat16
    ea = ea_ref[...]
    ad, qd = _unpack16(gd_ref[...])
    bs, ks = _unpack16(gs_ref[...])
    h1 = jnp.dot(ea.astype(bf), w1c_ref[...].astype(bf),
                 preferred_element_type=_F32)
    h1 = jnp.maximum(h1 + ad + bs + b1_ref[...], 0.0)
    h2 = jnp.dot(h1.astype(bf), w2_ref[...].astype(bf),
                 preferred_element_type=_F32) + b2_ref[...]
    mu = jnp.mean(h2, axis=-1, keepdims=True)
    dcen = h2 - mu
    var = jnp.mean(dcen * dcen, axis=-1, keepdims=True)
    e_new = g_ref[...] * dcen / jnp.sqrt(var + 1e-5) + b_ref[...]
    eout_ref[...] = ea + e_new
    prod = qd * ks
    w16 = jnp.exp(jnp.dot(prod.astype(bf), seg_ref[...].astype(bf),
                          preferred_element_type=_F32))
    wb = jnp.dot(w16.astype(bf), exp_ref[...].astype(bf),
                 preferred_element_type=_F32)
    wout_ref[...] = wb
    v = jnp.dot(e_new.astype(bf), wv_ref[...].astype(bf),
                preferred_element_type=_F32)
    pout_ref[...] = v * wb


def _edge(gd, gs, ea, w1c, b1, w2, b2, g, b, wv, seg16, exp16, ck, e_prev):
    cmat = lambda shape: pl.BlockSpec(shape, lambda i: (0, 0))
    in_specs = [
        pl.BlockSpec((_BE, _H), lambda i: (i, 0)),
        pl.BlockSpec((_BE, _H), lambda i: (i, 0)),
        pl.BlockSpec((_BE, _H), lambda i, c=ck: (i + c * _EBLK, 0)),
        cmat((_H, _H)), cmat((1, _H)), cmat((_H, _H)), cmat((1, _H)),
        cmat((1, _H)), cmat((1, _H)), cmat((_H, _H)),
        cmat((_H, 16)), cmat((16, _H)),
    ]
    args = [gd, gs, ea, w1c, b1, w2, b2, g, b, wv, seg16, exp16]
    aliases = {}
    body = _edge_body
    if e_prev is not None:
        in_specs.append(pl.BlockSpec((8, _H), lambda i: (0, 0)))
        args.append(e_prev)
        aliases = {12: 0}
        body = lambda *refs: _edge_body(*refs[:12], *refs[13:])
    return pl.pallas_call(
        body,
        grid=(_EBLK,),
        in_specs=in_specs,
        out_specs=[
            pl.BlockSpec((_BE, _H), lambda i, c=ck: (i + c * _EBLK, 0)),
            pl.BlockSpec((_BE, _H), lambda i: (i, 0)),
            pl.BlockSpec((_BE, _H), lambda i: (i, 0)),
        ],
        out_shape=[
            jax.ShapeDtypeStruct((_E, _H), _F32),
            jax.ShapeDtypeStruct((_EC, _H), _F32),
            jax.ShapeDtypeStruct((_EC, _H), _F32),
        ],
        input_output_aliases=aliases,
    )(*args)


def _make_scatter(cks):
    ncs = len(cks)

    @functools.partial(
        pl.kernel,
        mesh=_sc_mesh,
        out_type=[
            jax.ShapeDtypeStruct((_NC * _NP, _H), _F32),
            jax.ShapeDtypeStruct((_NC * _NP, _H), _F32),
        ],
        scratch_types=[
            pltpu.VMEM((_NSP, _CH), jnp.int32),
            pltpu.VMEM((_CH, _H), _F32),
            pltpu.VMEM((_CH, _H), _F32),
            pltpu.SemaphoreType.DMA,
            pltpu.SemaphoreType.DMA,
            pltpu.VMEM_SHARED((_NP, _H), _F32),
        ],
    )
    def scatter(*args):
        ps = args[:ncs]
        ws = args[ncs:2 * ncs]
        dst2_hbm = args[2 * ncs]
        macc_out, wacc_out, ixb, b0, b1, sl0, sl1, acc_sh = args[2 * ncs + 1:]
        cid = lax.axis_index("c")
        sid = lax.axis_index("s")
        wid = sid * _NC + cid
        slab = sid * _RPS
        nsub = _RPS // _CH
        obase = cid * _NP + slab
        zv = jnp.zeros((16,), _F32)
        bb = (b0, b1)
        sl = (sl0, sl1)

        def zero_vmem():
            def zbody(r, carry):
                for cc in range(_H // 16):
                    b0[r, pl.ds(cc * 16, 16)] = zv
                return carry

            lax.fori_loop(0, _CH, zbody, 0)

        def zero_slab():
            def ibody(j, carry):
                pltpu.sync_copy(b0, acc_sh.at[pl.ds(slab + j * _CH, _CH)])
                return carry

            lax.fori_loop(0, nsub, ibody, 0)

        def accumulate(srcs):
            for ck, src_hbm in zip(cks, srcs):
                cbase = wid * _EPWC
                pltpu.sync_copy(
                    dst2_hbm.at[pl.ds((ck * _NW + wid) * _NSP, _NSP)], ixb)

                def fire(i, p, src_hbm=src_hbm):
                    pltpu.async_copy(src_hbm.at[pl.ds(cbase + i * _CH, _CH)],
                                     bb[p], sl[p])

                def wait_rows(i, p, src_hbm=src_hbm):
                    pltpu.make_async_copy(
                        src_hbm.at[pl.ds(cbase + i * _CH, _CH)],
                        bb[p], sl[p]).wait()

                def add(i, p):
                    pltpu.sync_copy(bb[p], acc_sh.at[ixb.at[i]], add=True)

                fire(0, 0)

                def body(k, carry):
                    i0 = 2 * k
                    fire(i0 + 1, 1)
                    wait_rows(i0, 0)
                    add(i0, 0)

                    @pl.when(k < _NPAIR - 1)
                    def _():
                        fire(i0 + 2, 0)

                    wait_rows(i0 + 1, 1)
                    add(i0 + 1, 1)
                    return carry

                lax.fori_loop(0, _NPAIR, body, 0)

        def dump(out_hbm):
            def obody(j, carry):
                pltpu.sync_copy(acc_sh.at[pl.ds(slab + j * _CH, _CH)], b0)
                pltpu.sync_copy(b0, out_hbm.at[pl.ds(obase + j * _CH, _CH)])
                return carry

            lax.fori_loop(0, nsub, obody, 0)

        zero_vmem()
        zero_slab()
        plsc.subcore_barrier()
        accumulate(ps)
        plsc.subcore_barrier()
        dump(macc_out)
        zero_vmem()
        zero_slab()
        plsc.subcore_barrier()
        accumulate(ws)
        plsc.subcore_barrier()
        dump(wacc_out)

    return scatter


_sc_scatter_a = _make_scatter((0, 1, 2))
_sc_scatter_b = _make_scatter((3, 4))


def _node_body(x_ref, m_ref, wa_ref, mb_ref, wb_ref, wo_ref, nw1a_ref,
               nw1b_ref, nb1_ref, nw2_ref, nb2_ref, g_ref, b_ref, out_ref):
    xv = x_ref[...]
    macc = m_ref[0] + m_ref[1] + mb_ref[0] + mb_ref[1]
    wacc = wa_ref[0] + wa_ref[1] + wb_ref[0] + wb_ref[1]
    msg = macc / (wacc + 1e-12)
    msgo = jnp.dot(msg, wo_ref[...], preferred_element_type=_F32)
    h1 = (jnp.dot(xv, nw1a_ref[...], preferred_element_type=_F32)
          + jnp.dot(msgo, nw1b_ref[...], preferred_element_type=_F32)
          + nb1_ref[...])
    h1 = jnp.maximum(h1, 0.0)
    h2 = jnp.dot(h1, nw2_ref[...], preferred_element_type=_F32) + nb2_ref[...]
    mu = jnp.mean(h2, axis=-1, keepdims=True)
    dcen = h2 - mu
    var = jnp.mean(dcen * dcen, axis=-1, keepdims=True)
    out_ref[...] = xv + g_ref[...] * dcen / jnp.sqrt(var + 1e-5) + b_ref[...]


def _node(x, macc2, wacc2, macc2b, wacc2b, wo, nw1a, nw1b, nb1, nw2, nb2, g, b):
    cmat = lambda shape: pl.BlockSpec(shape, lambda i: (0, 0))
    return pl.pallas_call(
        _node_body,
        grid=(_N // _BN,),
        in_specs=[
            pl.BlockSpec((_BN, _H), lambda i: (i, 0)),
            pl.BlockSpec((2, _BN, _H), lambda i: (0, i, 0)),
            pl.BlockSpec((2, _BN, _H), lambda i: (0, i, 0)),
            pl.BlockSpec((2, _BN, _H), lambda i: (0, i, 0)),
            pl.BlockSpec((2, _BN, _H), lambda i: (0, i, 0)),
            cmat((_H, _H)), cmat((_H, _H)), cmat((_H, _H)), cmat((1, _H)),
            cmat((_H, _H)), cmat((1, _H)), cmat((1, _H)), cmat((1, _H)),
        ],
        out_specs=pl.BlockSpec((_BN, _H), lambda i: (i, 0)),
        out_shape=jax.ShapeDtypeStruct((_N, _H), _F32),
    )(x, macc2, wacc2, macc2b, wacc2b, wo, nw1a, nw1b, nb1, nw2, nb2, g, b)


def kernel(x, edge_index, edge_attr, edge_w1, edge_b1, edge_w2, edge_b2,
           edge_ln_g, edge_ln_b, node_w1, node_b1, node_w2, node_b2,
           node_ln_g, node_ln_b, wq, wk, wv, wo):
    src = edge_index[0].astype(jnp.int32)
    dst = edge_index[1].astype(jnp.int32)

    w1a = edge_w1[:_H]
    w1b = edge_w1[_H:2 * _H]
    w1c = edge_w1[2 * _H:]
    wd = jnp.concatenate([w1a, wq], axis=1)
    ws = jnp.concatenate([w1b, wk], axis=1)

    head_of = jnp.arange(_H, dtype=jnp.int32) // _DK
    lane = jnp.arange(16, dtype=jnp.int32)
    seg16 = (head_of[:, None] == lane[None, :]).astype(_F32) / math.sqrt(_DK)
    exp16 = (lane[:, None] == head_of[None, :]).astype(_F32)

    row = lambda v: v.reshape(1, _H)

    td, ts = _proj(x, wd, ws)

    def padded_idx(v):
        v4 = v.reshape(_NCK, _NW, _NSUBC, _CH)
        v4 = jnp.pad(v4, ((0, 0), (0, 0), (0, _NSP - _NSUBC), (0, 0)))
        return v4.reshape(_NCK, _NW * _NSP, _CH)

    dst2 = padded_idx(dst)
    src2 = padded_idx(src)

    p_cs, w_cs = [], []
    e_buf = None
    for ck in range(_NCK):
        gd, gs = _sc_gather(td, ts, dst2[ck], src2[ck])
        e_buf, p_c, w_c = _edge(gd, gs, edge_attr, w1c, row(edge_b1), edge_w2,
                                row(edge_b2), row(edge_ln_g), row(edge_ln_b),
                                wv, seg16, exp16, ck, e_buf)
        p_cs.append(p_c)
        w_cs.append(w_c)

    dst2f = dst2.reshape(_NCK * _NW * _NSP, _CH)
    macc_a, wacc_a = _sc_scatter_a(p_cs[0], p_cs[1], p_cs[2],
                                   w_cs[0], w_cs[1], w_cs[2], dst2f)
    macc_b, wacc_b = _sc_scatter_b(p_cs[3], p_cs[4], w_cs[3], w_cs[4], dst2f)
    x_new = _node(x, macc_a.reshape(2, _NP, _H), wacc_a.reshape(2, _NP, _H),
                  macc_b.reshape(2, _NP, _H), wacc_b.reshape(2, _NP, _H),
                  wo, node_w1[:_H], node_w1[_H:], row(node_b1), node_w2,
                  row(node_b2), row(node_ln_g), row(node_ln_b))
    return (x_new, e_buf)
